# baseline (speedup 1.0000x reference)
import jax
import jax.numpy as jnp
from jax import lax
from jax.experimental import pallas as pl
from jax.experimental.pallas import tpu as pltpu
from jax.experimental.pallas import tpu_sc as plsc

_N = 100000
_D = 128
_NW = 32
_CHUNK = 128
_NFULL = 24
_BIG = 3128
_SMALL = 3120
_TAIL_BIG = _BIG - _NFULL * _CHUNK
_TAIL_SMALL = _SMALL - _NFULL * _CHUNK
_NBUF = 6
_OUTER = _NFULL // _NBUF


def _embed_body(idx_hbm, table_hbm, out_hbm, idx_v, rows_v, table_sp, *sems):
    gsem = sems[:_NBUF]
    wsem = sems[_NBUF:]
    isem = sems[2 * _NBUF]
    sid = lax.axis_index("s")
    w = sid * 2 + lax.axis_index("c")
    big = w < 20
    rbase = _BIG * jnp.minimum(w, 20) + _SMALL * jnp.maximum(w - 20, 0)

    pltpu.async_copy(idx_hbm.at[pl.ds(rbase, _SMALL)],
                     idx_v.at[pl.ds(0, _SMALL)], isem)

    @pl.when(big)
    def _():
        pltpu.async_copy(idx_hbm.at[pl.ds(rbase + _SMALL, _BIG - _SMALL)],
                         idx_v.at[pl.ds(_SMALL, _BIG - _SMALL)], isem)

    @pl.when(sid < 14)
    def _():
        pltpu.sync_copy(table_hbm.at[pl.ds(sid * 8, 8)],
                        table_sp.at[pl.ds(sid * 8, 8)])

    @pl.when(sid == 14)
    def _():
        pltpu.sync_copy(table_hbm.at[pl.ds(112, 6)],
                        table_sp.at[pl.ds(112, 6)])

    plsc.subcore_barrier()

    pltpu.make_async_copy(idx_hbm.at[pl.ds(rbase, _SMALL)],
                          idx_v.at[pl.ds(0, _SMALL)], isem).wait()

    @pl.when(big)
    def _():
        pltpu.make_async_copy(
            idx_hbm.at[pl.ds(rbase + _SMALL, _BIG - _SMALL)],
            idx_v.at[pl.ds(_SMALL, _BIG - _SMALL)], isem).wait()

    def gather_start(j, s, src):
        pltpu.async_copy(
            src.at[idx_v.at[pl.ds(j * _CHUNK, _CHUNK)]],
            rows_v.at[s, pl.ds(0, _CHUNK)], gsem[s])

    def write_start(j, s, src):
        pltpu.make_async_copy(
            src.at[idx_v.at[pl.ds(j * _CHUNK, _CHUNK)]],
            rows_v.at[s, pl.ds(0, _CHUNK)], gsem[s]).wait()
        pltpu.async_copy(
            rows_v.at[s, pl.ds(0, _CHUNK)],
            out_hbm.at[pl.ds(rbase + j * _CHUNK, _CHUNK)], wsem[s])

    def write_wait(j, s):
        @pl.when(j >= 0)
        def _():
            pltpu.make_async_copy(
                rows_v.at[s, pl.ds(0, _CHUNK)],
                out_hbm.at[pl.ds(rbase + j * _CHUNK, _CHUNK)],
                wsem[s]).wait()

    def step(k, carry):
        for s in range(_NBUF):
            j = k * _NBUF + s
            src = table_hbm if s == 2 else table_sp
            write_wait(j - _NBUF, s)
            gather_start(j, s, src)
        for s in range(_NBUF):
            j = k * _NBUF + s
            src = table_hbm if s == 2 else table_sp
            write_start(j, s, src)
        return carry

    lax.fori_loop(0, _OUTER, step, 0)

    toff = _NFULL * _CHUNK
    write_wait(_NFULL - _NBUF, 0)

    def tail(tsz):
        pltpu.async_copy(
            table_sp.at[idx_v.at[pl.ds(toff, tsz)]],
            rows_v.at[0, pl.ds(0, tsz)], gsem[0])
        pltpu.make_async_copy(
            table_sp.at[idx_v.at[pl.ds(toff, tsz)]],
            rows_v.at[0, pl.ds(0, tsz)], gsem[0]).wait()
        pltpu.async_copy(
            rows_v.at[0, pl.ds(0, tsz)],
            out_hbm.at[pl.ds(rbase + toff, tsz)], wsem[0])
        pltpu.make_async_copy(
            rows_v.at[0, pl.ds(0, tsz)],
            out_hbm.at[pl.ds(rbase + toff, tsz)], wsem[0]).wait()

    @pl.when(big)
    def _():
        tail(_TAIL_BIG)

    @pl.when(jnp.logical_not(big))
    def _():
        tail(_TAIL_SMALL)

    for s in range(1, _NBUF):
        write_wait(_NFULL - _NBUF + s, s)


def kernel(atomic_numbers, table):
    idx = atomic_numbers.astype(jnp.int32)
    mesh = plsc.VectorSubcoreMesh(core_axis_name="c", subcore_axis_name="s")
    f = pl.kernel(
        _embed_body,
        out_type=jax.ShapeDtypeStruct((_N, _D), jnp.float32),
        scratch_types=[
            pltpu.VMEM((_BIG,), jnp.int32),
            pltpu.VMEM((_NBUF, _CHUNK, _D), jnp.float32),
            pltpu.VMEM_SHARED((118, _D), jnp.float32),
        ] + [pltpu.SemaphoreType.DMA] * (2 * _NBUF + 1),
        mesh=mesh,
    )
    return f(idx, table)


# speedup vs baseline: 1.5395x; 1.5395x over previous
import jax
import jax.numpy as jnp
from jax import lax
from jax.experimental import pallas as pl
from jax.experimental.pallas import tpu as pltpu
from jax.experimental.pallas import tpu_sc as plsc

_N = 100000
_D = 128
_NW = 32
_CHUNK = 128
_NFULL = 24
_BIG = 3128
_SMALL = 3120
_TAIL_BIG = _BIG - _NFULL * _CHUNK
_TAIL_SMALL = _SMALL - _NFULL * _CHUNK
_NBUF = 6
_OUTER = _NFULL // _NBUF


def _embed_body(idx_hbm, table_hbm, out_hbm, idx_v, rows_v, table_sp, *sems):
    gsem = sems[:_NBUF]
    wsem = sems[_NBUF:]
    isem = sems[2 * _NBUF]
    sid = lax.axis_index("s")
    w = sid * 2 + lax.axis_index("c")
    big = w < 20
    rbase = _BIG * jnp.minimum(w, 20) + _SMALL * jnp.maximum(w - 20, 0)

    pltpu.async_copy(idx_hbm.at[pl.ds(rbase, _SMALL)],
                     idx_v.at[pl.ds(0, _SMALL)], isem)

    @pl.when(big)
    def _():
        pltpu.async_copy(idx_hbm.at[pl.ds(rbase + _SMALL, _BIG - _SMALL)],
                         idx_v.at[pl.ds(_SMALL, _BIG - _SMALL)], isem)

    @pl.when(sid < 14)
    def _():
        pltpu.sync_copy(table_hbm.at[pl.ds(sid * 8, 8)],
                        table_sp.at[pl.ds(sid * 8, 8)])

    @pl.when(sid == 14)
    def _():
        pltpu.sync_copy(table_hbm.at[pl.ds(112, 6)],
                        table_sp.at[pl.ds(112, 6)])

    plsc.subcore_barrier()

    pltpu.make_async_copy(idx_hbm.at[pl.ds(rbase, _SMALL)],
                          idx_v.at[pl.ds(0, _SMALL)], isem).wait()

    @pl.when(big)
    def _():
        pltpu.make_async_copy(
            idx_hbm.at[pl.ds(rbase + _SMALL, _BIG - _SMALL)],
            idx_v.at[pl.ds(_SMALL, _BIG - _SMALL)], isem).wait()

    def gather_start(j, s, src):
        pltpu.async_copy(
            src.at[idx_v.at[pl.ds(j * _CHUNK, _CHUNK)]],
            rows_v.at[s, pl.ds(0, _CHUNK)], gsem[s])

    def write_start(j, s, src):
        pltpu.make_async_copy(
            src.at[idx_v.at[pl.ds(j * _CHUNK, _CHUNK)]],
            rows_v.at[s, pl.ds(0, _CHUNK)], gsem[s]).wait()
        pltpu.async_copy(
            rows_v.at[s, pl.ds(0, _CHUNK)],
            out_hbm.at[pl.ds(rbase + j * _CHUNK, _CHUNK)], wsem[s])

    def write_wait(j, s):
        @pl.when(j >= 0)
        def _():
            pltpu.make_async_copy(
                rows_v.at[s, pl.ds(0, _CHUNK)],
                out_hbm.at[pl.ds(rbase + j * _CHUNK, _CHUNK)],
                wsem[s]).wait()

    def step(k, carry):
        for s in range(_NBUF):
            j = k * _NBUF + s
            write_wait(j - _NBUF, s)
            gather_start(j, s, table_sp)
        for s in range(_NBUF):
            j = k * _NBUF + s
            write_start(j, s, table_sp)
        return carry

    lax.fori_loop(0, _OUTER, step, 0)

    toff = _NFULL * _CHUNK
    write_wait(_NFULL - _NBUF, 0)

    def tail(tsz):
        pltpu.async_copy(
            table_sp.at[idx_v.at[pl.ds(toff, tsz)]],
            rows_v.at[0, pl.ds(0, tsz)], gsem[0])
        pltpu.make_async_copy(
            table_sp.at[idx_v.at[pl.ds(toff, tsz)]],
            rows_v.at[0, pl.ds(0, tsz)], gsem[0]).wait()
        pltpu.async_copy(
            rows_v.at[0, pl.ds(0, tsz)],
            out_hbm.at[pl.ds(rbase + toff, tsz)], wsem[0])
        pltpu.make_async_copy(
            rows_v.at[0, pl.ds(0, tsz)],
            out_hbm.at[pl.ds(rbase + toff, tsz)], wsem[0]).wait()

    @pl.when(big)
    def _():
        tail(_TAIL_BIG)

    @pl.when(jnp.logical_not(big))
    def _():
        tail(_TAIL_SMALL)

    for s in range(1, _NBUF):
        write_wait(_NFULL - _NBUF + s, s)


def kernel(atomic_numbers, table):
    idx = atomic_numbers.astype(jnp.int32)
    mesh = plsc.VectorSubcoreMesh(core_axis_name="c", subcore_axis_name="s")
    f = pl.kernel(
        _embed_body,
        out_type=jax.ShapeDtypeStruct((_N, _D), jnp.float32),
        scratch_types=[
            pltpu.VMEM((_BIG,), jnp.int32),
            pltpu.VMEM((_NBUF, _CHUNK, _D), jnp.float32),
            pltpu.VMEM_SHARED((118, _D), jnp.float32),
        ] + [pltpu.SemaphoreType.DMA] * (2 * _NBUF + 1),
        mesh=mesh,
    )
    return f(idx, table)
